# baseline (speedup 1.0000x reference)
import jax
import jax.numpy as jnp
from jax import lax
from jax.experimental import pallas as pl
from jax.experimental.pallas import tpu as pltpu
from jax.experimental.pallas import tpu_sc as plsc

B = 512
D = 114688
V = 128
NCH = 64
NCHS = NCH // 2
DC = D // NCH
NS = 16
RPT = B // NS
VPT = V // NS
NSP = 20
NTS = NCHS - NSP
NB = NTS - NSP // 2


def _body(pf_hbm, tbl_hbm, out_hbm, idx_v, tb0, tb1, sp_a, sp_b,
          stg_a, stg_b, dsem, gsem, ws0, ws1):
    core = lax.axis_index("c")
    s = lax.axis_index("s")
    tout = s * RPT
    pltpu.sync_copy(pf_hbm.at[pl.ds(tout, RPT)], idx_v)
    regs = [idx_v[pl.ds(0, 16)], idx_v[pl.ds(16, 16)]]
    rows = [regs[g][j] for g in range(2) for j in range(16)]
    tbs = [tb0, tb1]
    wss = [ws0, ws1]

    def col(c):
        return (core * NCHS + c) * DC

    def stage(c, buf, sem):
        return pltpu.make_async_copy(
            tbl_hbm.at[pl.ds(s * VPT, VPT), pl.ds(col(c), DC)],
            buf.at[pl.ds(s * VPT, VPT), :], sem)

    def fanout(c, buf):
        return [
            pltpu.make_async_copy(
                buf.at[rows[j]], out_hbm.at[tout + j, pl.ds(col(c), DC)],
                dsem)
            for j in range(RPT)
        ]

    def ts_gather(c, g):
        return pltpu.make_async_copy(
            tbl_hbm.at[regs[g], pl.ds(col(c), DC)], tbs[g], gsem)

    def ts_write(c, g):
        return pltpu.make_async_copy(
            tbs[g],
            out_hbm.at[pl.ds(tout + g * 16, 16), pl.ds(col(c), DC)], wss[g])

    def ts_chunk(c3):
        for g in range(2):
            ts_gather(c3, g).start()
            ts_gather(c3, g).wait()
            ts_write(c3, g).start()

    def ts_drain(c3):
        for g in range(2):
            ts_write(c3, g).wait()

    last = jnp.int32(NSP - 1)
    stage(0, sp_a, stg_a).start()
    stage(1, sp_b, stg_b).start()

    def step(i, carry):
        c0 = 2 * i
        c1 = 2 * i + 1
        c3a = NSP + NB + i
        c3b = NSP + i
        stage(c0, sp_a, stg_a).wait()
        plsc.subcore_barrier()
        cps = fanout(c0, sp_a)
        for cp in cps:
            cp.start()
        @pl.when(i > 0)
        def _():
            ts_drain(c3a)
        ts_chunk(c3a)
        for cp in cps:
            cp.wait()
        plsc.subcore_barrier()
        stage(jnp.minimum(c0 + 2, last), sp_a, stg_a).start()

        stage(c1, sp_b, stg_b).wait()
        plsc.subcore_barrier()
        cps = fanout(c1, sp_b)
        for cp in cps:
            cp.start()
        @pl.when(i < NB)
        def _():
            ts_drain(c3b)
            ts_chunk(c3b)
        for cp in cps:
            cp.wait()
        plsc.subcore_barrier()
        stage(jnp.minimum(c1 + 2, last), sp_b, stg_b).start()
        return carry

    lax.fori_loop(0, NSP // 2, step, 0)
    stage(last, sp_a, stg_a).wait()
    stage(last, sp_b, stg_b).wait()
    ts_drain(NCHS - 1)


_gather = pl.kernel(
    _body,
    out_type=jax.ShapeDtypeStruct((B, D), jnp.float32),
    mesh=plsc.VectorSubcoreMesh(core_axis_name="c", subcore_axis_name="s"),
    scratch_types=[
        pltpu.VMEM((RPT,), jnp.int32),
        pltpu.VMEM((16, DC), jnp.float32),
        pltpu.VMEM((16, DC), jnp.float32),
        pltpu.VMEM_SHARED((V, DC), jnp.float32),
        pltpu.VMEM_SHARED((V, DC), jnp.float32),
        pltpu.SemaphoreType.DMA,
        pltpu.SemaphoreType.DMA,
        pltpu.SemaphoreType.DMA,
        pltpu.SemaphoreType.DMA,
        pltpu.SemaphoreType.DMA,
        pltpu.SemaphoreType.DMA,
    ],
)


def kernel(prefix, embedding_table):
    pf = prefix.reshape(B).astype(jnp.int32)
    out = _gather(pf, embedding_table)
    return out.reshape(prefix.shape[0], prefix.shape[1], D)


# speedup vs baseline: 1.1177x; 1.1177x over previous
import jax
import jax.numpy as jnp
from jax import lax
from jax.experimental import pallas as pl
from jax.experimental.pallas import tpu as pltpu
from jax.experimental.pallas import tpu_sc as plsc

B = 512
D = 114688
V = 128
NCH = 64
NCHS = NCH // 2
DC = D // NCH
NS = 16
RPT = B // NS
VPT = V // NS
NSP = 22
NTS = NCHS - NSP


def _body(pf_hbm, tbl_hbm, out_hbm, idx_v, tb0, tb1, sp_a, sp_b,
          stg_a, stg_b, dsem, gsem, ws0, ws1):
    core = lax.axis_index("c")
    s = lax.axis_index("s")
    tout = s * RPT
    pltpu.sync_copy(pf_hbm.at[pl.ds(tout, RPT)], idx_v)
    regs = [idx_v[pl.ds(0, 16)], idx_v[pl.ds(16, 16)]]
    rows = [regs[g][j] for g in range(2) for j in range(16)]
    tbs = [tb0, tb1]
    wss = [ws0, ws1]

    def col(c):
        return (core * NCHS + c) * DC

    def stage(c, buf, sem):
        return pltpu.make_async_copy(
            tbl_hbm.at[pl.ds(s * VPT, VPT), pl.ds(col(c), DC)],
            buf.at[pl.ds(s * VPT, VPT), :], sem)

    def fanout(c, buf):
        return [
            pltpu.make_async_copy(
                buf.at[rows[j]], out_hbm.at[tout + j, pl.ds(col(c), DC)],
                dsem)
            for j in range(RPT)
        ]

    def ts_gather(c, g):
        return pltpu.make_async_copy(
            tbl_hbm.at[regs[g], pl.ds(col(c), DC)], tbs[g], gsem)

    def ts_write(c, g):
        return pltpu.make_async_copy(
            tbs[g],
            out_hbm.at[pl.ds(tout + g * 16, 16), pl.ds(col(c), DC)], wss[g])

    def ts_half(i, g):
        c3 = NSP + i
        @pl.when(jnp.logical_and(i > 0, i < NTS))
        def _():
            ts_write(c3, g).wait()
        @pl.when(i < NTS)
        def _():
            ts_gather(c3, g).start()
            ts_gather(c3, g).wait()
            ts_write(c3, g).start()

    last = jnp.int32(NSP - 1)
    stage(0, sp_a, stg_a).start()
    stage(1, sp_b, stg_b).start()

    def step(i, carry):
        c0 = 2 * i
        c1 = 2 * i + 1
        stage(c0, sp_a, stg_a).wait()
        plsc.subcore_barrier()
        cps = fanout(c0, sp_a)
        for cp in cps:
            cp.start()
        ts_half(i, 0)
        for cp in cps:
            cp.wait()
        plsc.subcore_barrier()
        stage(jnp.minimum(c0 + 2, last), sp_a, stg_a).start()

        stage(c1, sp_b, stg_b).wait()
        plsc.subcore_barrier()
        cps = fanout(c1, sp_b)
        for cp in cps:
            cp.start()
        ts_half(i, 1)
        for cp in cps:
            cp.wait()
        plsc.subcore_barrier()
        stage(jnp.minimum(c1 + 2, last), sp_b, stg_b).start()
        return carry

    lax.fori_loop(0, NSP // 2, step, 0)
    stage(last, sp_a, stg_a).wait()
    stage(last, sp_b, stg_b).wait()
    ts_write(NCHS - 1, 0).wait()
    ts_write(NCHS - 1, 1).wait()


_gather = pl.kernel(
    _body,
    out_type=jax.ShapeDtypeStruct((B, D), jnp.float32),
    mesh=plsc.VectorSubcoreMesh(core_axis_name="c", subcore_axis_name="s"),
    scratch_types=[
        pltpu.VMEM((RPT,), jnp.int32),
        pltpu.VMEM((16, DC), jnp.float32),
        pltpu.VMEM((16, DC), jnp.float32),
        pltpu.VMEM_SHARED((V, DC), jnp.float32),
        pltpu.VMEM_SHARED((V, DC), jnp.float32),
        pltpu.SemaphoreType.DMA,
        pltpu.SemaphoreType.DMA,
        pltpu.SemaphoreType.DMA,
        pltpu.SemaphoreType.DMA,
        pltpu.SemaphoreType.DMA,
        pltpu.SemaphoreType.DMA,
    ],
)


def kernel(prefix, embedding_table):
    pf = prefix.reshape(B).astype(jnp.int32)
    out = _gather(pf, embedding_table)
    return out.reshape(prefix.shape[0], prefix.shape[1], D)


# speedup vs baseline: 1.1209x; 1.0029x over previous
import jax
import jax.numpy as jnp
from jax import lax
from jax.experimental import pallas as pl
from jax.experimental.pallas import tpu as pltpu
from jax.experimental.pallas import tpu_sc as plsc

B = 512
D = 114688
V = 128
NCH = 64
NCHS = NCH // 2
DC = D // NCH
NS = 16
RPT = B // NS
VPT = V // NS
NSP = 22
NTS = NCHS - NSP


def _body(pf_hbm, tbl_hbm, out_hbm, idx_v, tb0, tb1, sp_a, sp_b,
          stg_a, stg_b, dsem, gsem, ws0, ws1):
    core = lax.axis_index("c")
    s = lax.axis_index("s")
    tout = s * RPT
    tbs = [tb0, tb1]
    wss = [ws0, ws1]

    def col(c):
        return (core * NCHS + c) * DC

    def stage(c, buf, sem):
        return pltpu.make_async_copy(
            tbl_hbm.at[pl.ds(s * VPT, VPT), pl.ds(col(c), DC)],
            buf.at[pl.ds(s * VPT, VPT), :], sem)

    def fanout(c, buf):
        return [
            pltpu.make_async_copy(
                buf.at[rows[j]], out_hbm.at[tout + j, pl.ds(col(c), DC)],
                dsem)
            for j in range(RPT)
        ]

    def ts_gather(c, g):
        return pltpu.make_async_copy(
            tbl_hbm.at[regs[g], pl.ds(col(c), DC)], tbs[g], gsem)

    def ts_write(c, g):
        return pltpu.make_async_copy(
            tbs[g],
            out_hbm.at[pl.ds(tout + g * 16, 16), pl.ds(col(c), DC)], wss[g])

    def ts_half(i, g):
        c3 = NSP + i
        @pl.when(jnp.logical_and(i > 0, i < NTS))
        def _():
            ts_write(c3, g).wait()
        @pl.when(i < NTS)
        def _():
            ts_gather(c3, g).start()
            ts_gather(c3, g).wait()
            ts_write(c3, g).start()

    last = jnp.int32(NSP - 1)
    stage(0, sp_a, stg_a).start()
    stage(1, sp_b, stg_b).start()
    pltpu.sync_copy(pf_hbm.at[s // 4, pl.ds((s % 4) * RPT, RPT)], idx_v)
    regs = [idx_v[pl.ds(0, 16)], idx_v[pl.ds(16, 16)]]
    rows = [regs[g][j] for g in range(2) for j in range(16)]

    def step(i, carry):
        c0 = 2 * i
        c1 = 2 * i + 1
        stage(c0, sp_a, stg_a).wait()
        plsc.subcore_barrier()
        cps = fanout(c0, sp_a)
        for cp in cps:
            cp.start()
        ts_half(i, 0)
        for cp in cps:
            cp.wait()
        plsc.subcore_barrier()
        stage(jnp.minimum(c0 + 2, last), sp_a, stg_a).start()

        stage(c1, sp_b, stg_b).wait()
        plsc.subcore_barrier()
        cps = fanout(c1, sp_b)
        for cp in cps:
            cp.start()
        ts_half(i, 1)
        for cp in cps:
            cp.wait()
        plsc.subcore_barrier()
        stage(jnp.minimum(c1 + 2, last), sp_b, stg_b).start()
        return carry

    lax.fori_loop(0, NSP // 2, step, 0)
    stage(last, sp_a, stg_a).wait()
    stage(last, sp_b, stg_b).wait()
    ts_write(NCHS - 1, 0).wait()
    ts_write(NCHS - 1, 1).wait()


_gather = pl.kernel(
    _body,
    out_type=jax.ShapeDtypeStruct((B, D), jnp.float32),
    mesh=plsc.VectorSubcoreMesh(core_axis_name="c", subcore_axis_name="s"),
    scratch_types=[
        pltpu.VMEM((RPT,), jnp.int32),
        pltpu.VMEM((16, DC), jnp.float32),
        pltpu.VMEM((16, DC), jnp.float32),
        pltpu.VMEM_SHARED((V, DC), jnp.float32),
        pltpu.VMEM_SHARED((V, DC), jnp.float32),
        pltpu.SemaphoreType.DMA,
        pltpu.SemaphoreType.DMA,
        pltpu.SemaphoreType.DMA,
        pltpu.SemaphoreType.DMA,
        pltpu.SemaphoreType.DMA,
        pltpu.SemaphoreType.DMA,
    ],
)


def kernel(prefix, embedding_table):
    pf = prefix.astype(jnp.int32)
    out = _gather(pf, embedding_table)
    return out.reshape(prefix.shape[0], prefix.shape[1], D)
